# baseline (speedup 1.0000x reference)
import functools

import jax
import jax.numpy as jnp
from jax import lax
from jax.experimental import pallas as pl
from jax.experimental.pallas import tpu as pltpu
from jax.experimental.pallas import tpu_sc as plsc

B = 4096
A = 200
A_PAD = 208
A_OUT = 256
ED = 128
HD = 256
AD = ED + HD // 2
NR_PAD = 512
HUGE = 1e9

NC, NS, L = 2, 16, 16
NW = NC * NS
BPW = B // NW
CH = 8
NCHUNK = BPW // CH
GH = A_PAD // 2

@functools.cache
def _build_gather_e():
    mesh = plsc.VectorSubcoreMesh(core_axis_name="c", subcore_axis_name="s",
                                  num_cores=NC, num_subcores=NS)

    @functools.partial(
        pl.kernel,
        out_type=jax.ShapeDtypeStruct((B, ED), jnp.float32),
        mesh=mesh,
        scratch_types=[
            pltpu.VMEM((BPW,), jnp.int32),
            pltpu.VMEM((BPW, ED), jnp.float32),
            pltpu.SemaphoreType.DMA,
        ],
        compiler_params=pltpu.CompilerParams(needs_layout_passes=False, use_tc_tiling_on_sc=False),
    )
    def _gather_e(table_hbm, idx_hbm, out_hbm, idx_v, rows_v, sem):
        wid = lax.axis_index("s") * NC + lax.axis_index("c")
        base = wid * BPW
        pltpu.sync_copy(idx_hbm.at[pl.ds(base, BPW)], idx_v)
        pltpu.async_copy(table_hbm.at[idx_v], rows_v, sem).wait()
        pltpu.sync_copy(rows_v, out_hbm.at[pl.ds(base, BPW)])

    return _gather_e


def _mlp_body(e_ref, h_ref, w1_ref, b1_ref, w2_ref, b2_ref, relT_ref,
              x2e_ref, rel_ref):
    dot = functools.partial(
        jax.lax.dot_general,
        dimension_numbers=(((1,), (0,)), ((), ())),
        preferred_element_type=jnp.float32,
        precision=jax.lax.Precision.HIGHEST,
    )
    x = dot(e_ref[...], w1_ref[:ED, :]) + dot(h_ref[...], w1_ref[ED:, :])
    x = jnp.maximum(x + b1_ref[...], 0.0)
    x2 = jnp.maximum(dot(x, w2_ref[...]) + b2_ref[...], 0.0)
    x2e_ref[...] = x2[:, ED:]
    rel_ref[...] = dot(x2[:, :ED], relT_ref[...])


def _mlp(E, H, W1, b1, W2, b2, relT):
    bs = 512
    grid = (B // bs,)
    return pl.pallas_call(
        _mlp_body,
        grid=grid,
        in_specs=[
            pl.BlockSpec((bs, ED), lambda i: (i, 0)),
            pl.BlockSpec((bs, HD), lambda i: (i, 0)),
            pl.BlockSpec((ED + HD, AD), lambda i: (0, 0)),
            pl.BlockSpec((1, AD), lambda i: (0, 0)),
            pl.BlockSpec((AD, AD), lambda i: (0, 0)),
            pl.BlockSpec((1, AD), lambda i: (0, 0)),
            pl.BlockSpec((ED, NR_PAD), lambda i: (0, 0)),
        ],
        out_specs=[
            pl.BlockSpec((bs, ED), lambda i: (i, 0)),
            pl.BlockSpec((bs, NR_PAD), lambda i: (i, 0)),
        ],
        out_shape=[
            jax.ShapeDtypeStruct((B, ED), jnp.float32),
            jax.ShapeDtypeStruct((B, NR_PAD), jnp.float32),
        ],
    )(E, H, W1, b1, W2, b2, relT)


@functools.cache
def _build_scores_sc():
    mesh = plsc.VectorSubcoreMesh(core_axis_name="c", subcore_axis_name="s",
                                  num_cores=NC, num_subcores=NS)

    @functools.partial(
        pl.kernel,
        out_type=jax.ShapeDtypeStruct((B, A_OUT), jnp.float32),
        mesh=mesh,
        scratch_types=[
            pltpu.VMEM((2, CH, A_PAD), jnp.int32),
            pltpu.VMEM((2, CH, A_PAD), jnp.int32),
            pltpu.VMEM((2, CH, A_PAD), jnp.float32),
            pltpu.VMEM((BPW, ED), jnp.float32),
            pltpu.VMEM((2, CH, NR_PAD), jnp.float32),
            pltpu.VMEM((2, A_PAD, ED // 2), jnp.int32),
            pltpu.VMEM((2, CH, A_OUT), jnp.float32),
            pltpu.SemaphoreType.DMA,
            pltpu.SemaphoreType.DMA,
            pltpu.SemaphoreType.DMA,
            pltpu.SemaphoreType.DMA,
            pltpu.SemaphoreType.DMA,
            pltpu.SemaphoreType.DMA,
        ],
        compiler_params=pltpu.CompilerParams(needs_layout_passes=False, use_tc_tiling_on_sc=False),
    )
    def _scores_sc(esp_hbm, rsp_hbm, msk_hbm, x2e_hbm, rel_hbm, table_hbm,
                   out_hbm, eidx_v, ridx_v, msk_v, x2e_v, rel_v, rows_v, sc_v,
                   csem0, csem1, gsem0, gsem1, osem0, osem1):
        _scores_body(esp_hbm, rsp_hbm, msk_hbm, x2e_hbm, rel_hbm, table_hbm,
                     out_hbm, eidx_v, ridx_v, msk_v, x2e_v, rel_v, rows_v,
                     sc_v, (csem0, csem1), (gsem0, gsem1), (osem0, osem1))

    return _scores_sc


def _scores_body(esp_hbm, rsp_hbm, msk_hbm, x2e_hbm, rel_hbm, table_hbm,
                 out_hbm, eidx_v, ridx_v, msk_v, x2e_v, rel_v, rows_v, sc_v,
                 csems, gsems, osems):
    wid = lax.axis_index("s") * NC + lax.axis_index("c")
    base = wid * BPW
    lane = lax.iota(jnp.int32, L)
    perms = tuple(lane ^ sh for sh in (8, 4, 2, 1))

    zi = jnp.zeros((L,), jnp.int32)
    zf = jnp.zeros((L,), jnp.float32)
    neg = jnp.full((L,), -1e30, jnp.float32)
    pad_idx = wid * L + lane
    for slot in range(2):
        for bb in range(CH):
            eidx_v[slot, bb, pl.ds(192, L)] = pad_idx
            ridx_v[slot, bb, pl.ds(192, L)] = zi
            msk_v[slot, bb, pl.ds(192, L)] = zf
            for g in (13, 14, 15):
                sc_v[slot, bb, pl.ds(g * L, L)] = neg

    def issue_chunk(slot, c):
        cb = base + c * CH
        for bb in range(CH):
            pltpu.async_copy(esp_hbm.at[cb + bb],
                             eidx_v.at[slot, bb, pl.ds(0, A)], csems[slot])
            pltpu.async_copy(rsp_hbm.at[cb + bb],
                             ridx_v.at[slot, bb, pl.ds(0, A)], csems[slot])
            pltpu.async_copy(msk_hbm.at[cb + bb],
                             msk_v.at[slot, bb, pl.ds(0, A)], csems[slot])
        pltpu.async_copy(rel_hbm.at[pl.ds(cb, CH)], rel_v.at[slot],
                         csems[slot])

    def wait_chunk(slot):
        for bb in range(CH):
            pltpu.make_async_copy(esp_hbm.at[base],
                                  eidx_v.at[slot, bb, pl.ds(0, A)],
                                  csems[slot]).wait()
            pltpu.make_async_copy(rsp_hbm.at[base],
                                  ridx_v.at[slot, bb, pl.ds(0, A)],
                                  csems[slot]).wait()
            pltpu.make_async_copy(msk_hbm.at[base],
                                  msk_v.at[slot, bb, pl.ds(0, A)],
                                  csems[slot]).wait()
        pltpu.make_async_copy(rel_hbm.at[pl.ds(base, CH)], rel_v.at[slot],
                              csems[slot]).wait()

    QPARTS = ((0, 32), (32, 24), (56, 24), (80, 24),
              (104, 32), (136, 24), (160, 24), (184, 24))

    def issue_gather(islot, brow, bslot):
        for off, n in QPARTS:
            pltpu.async_copy(table_hbm.at[eidx_v.at[islot, brow, pl.ds(off, n)]],
                             rows_v.at[bslot, pl.ds(off, n)], gsems[bslot])

    def wait_gather(bslot):
        for off, n in QPARTS:
            pltpu.make_async_copy(table_hbm.at[pl.ds(0, n)],
                                  rows_v.at[bslot, pl.ds(off, n)],
                                  gsems[bslot]).wait()

    def issue_out(slot, c):
        pltpu.async_copy(sc_v.at[slot], out_hbm.at[pl.ds(base + c * CH, CH)],
                         osems[slot])

    def wait_out(slot):
        pltpu.make_async_copy(sc_v.at[slot], out_hbm.at[pl.ds(base, CH)],
                              osems[slot]).wait()

    def _lane_sum(v):
        for p in perms:
            v = v + v.at[p].get(mode="promise_in_bounds")
        return v

    def compute(p, q, bb, b_local):
        xk = tuple(x2e_v[b_local, pl.ds(k * L, L)] for k in range(ED // L))

        def group_body(g, c):
            a0 = g * L
            res = jnp.zeros((L,), jnp.float32)
            for j in range(L):
                a = a0 + j
                acc = jnp.zeros((L,), jnp.float32)
                for cc in range(ED // (2 * L)):
                    w = rows_v[q, a, pl.ds(cc * L, L)]
                    bf = plsc.bitcast(w, jnp.bfloat16)
                    ua, ub = plsc.unpack(bf, format=plsc.PackFormat.INTERLEAVED)
                    acc = acc + ua * xk[2 * cc] + ub * xk[2 * cc + 1]
                res = jnp.where(lane == j, _lane_sum(acc), res)
            ri = ridx_v[p, bb, pl.ds(a0, L)]
            rv = plsc.load_gather(rel_v.at[p],
                                  [jnp.full((L,), bb, jnp.int32), ri])
            mv = msk_v[p, bb, pl.ds(a0, L)]
            sc_v[p, bb, pl.ds(a0, L)] = res + rv - (1.0 - mv) * HUGE
            return c

        lax.fori_loop(0, A_PAD // L, group_body, 0)

    pltpu.sync_copy(x2e_hbm.at[pl.ds(base, BPW)], x2e_v)
    issue_chunk(0, 0)
    wait_chunk(0)
    issue_gather(0, 0, 0)

    @pl.loop(0, NCHUNK, step=2)
    def chunk_loop(ci):
        for p in range(2):
            c = ci + p
            cnext = jnp.minimum(c + 1, NCHUNK - 1)

            @pl.when(c >= 2)
            def _():
                wait_out(p)

            issue_chunk(p ^ 1, cnext)

            @pl.loop(0, CH, step=2)
            def b_loop(qi):
                for q in range(2):
                    bb = qi + q
                    wait_gather(q)

                    @pl.when(bb < CH - 1)
                    def _():
                        issue_gather(p, bb + 1, q ^ 1)

                    @pl.when(bb == CH - 1)
                    def _():
                        wait_chunk(p ^ 1)
                        issue_gather(p ^ 1, 0, q ^ 1)

                    compute(p, q, bb, c * CH + bb)

            issue_out(p, c)

    wait_gather(0)
    wait_out(0)
    wait_out(1)


def _soft_body(s_ref, dist_ref, ent_ref):
    s = s_ref[...]
    m = jnp.max(s, axis=1, keepdims=True)
    ex = jnp.exp(s - m)
    z = jnp.sum(ex, axis=1, keepdims=True)
    p = ex / z
    dist_ref[...] = p
    ent_ref[...] = -jnp.sum(p * jnp.log(p + 1e-20), axis=1, keepdims=True)


def _softmax(scores):
    bs = 512
    return pl.pallas_call(
        _soft_body,
        grid=(B // bs,),
        in_specs=[pl.BlockSpec((bs, A_OUT), lambda i: (i, 0))],
        out_specs=[
            pl.BlockSpec((bs, A_OUT), lambda i: (i, 0)),
            pl.BlockSpec((bs, 1), lambda i: (i, 0)),
        ],
        out_shape=[
            jax.ShapeDtypeStruct((B, A_OUT), jnp.float32),
            jax.ShapeDtypeStruct((B, 1), jnp.float32),
        ],
    )(scores)


def kernel(e, H, r_space, e_space, action_mask, entity_emb, relation_emb,
           W1, b1, W2, b2):
    e = e.astype(jnp.int32)
    r_space = r_space.astype(jnp.int32)
    e_space = e_space.astype(jnp.int32)
    nr1 = relation_emb.shape[0]
    relT = jnp.zeros((ED, NR_PAD), jnp.float32).at[:, :nr1].set(relation_emb.T)

    E = _build_gather_e()(entity_emb, e)
    x2e, rel_scores = _mlp(E, H, W1, b1.reshape(1, AD), W2, b2.reshape(1, AD),
                           relT)
    ent16 = lax.bitcast_convert_type(
        entity_emb.astype(jnp.bfloat16).reshape(-1, ED // 2, 2), jnp.int32)
    x2e_de = x2e.reshape(B, ED // (2 * L), L, 2)
    x2e_de = x2e_de.transpose(0, 1, 3, 2).reshape(B, ED)
    scores = _build_scores_sc()(e_space, r_space, action_mask, x2e_de,
                                rel_scores, ent16)
    dist, ent = _softmax(scores)
    return dist[:, :A], ent.reshape(B)


# speedup vs baseline: 2.2168x; 2.2168x over previous
import functools

import jax
import jax.numpy as jnp
from jax import lax
from jax.experimental import pallas as pl
from jax.experimental.pallas import tpu as pltpu
from jax.experimental.pallas import tpu_sc as plsc

B = 4096
A = 200
A_PAD = 208
A_OUT = 256
ED = 128
HD = 256
AD = ED + HD // 2
NR_PAD = 512
HUGE = 1e9

NC, NS, L = 2, 16, 16
NW = NC * NS
BPW = B // NW
CH = 8
NCHUNK = BPW // CH
GH = A_PAD // 2

@functools.cache
def _build_gather_e():
    mesh = plsc.VectorSubcoreMesh(core_axis_name="c", subcore_axis_name="s",
                                  num_cores=NC, num_subcores=NS)

    @functools.partial(
        pl.kernel,
        out_type=jax.ShapeDtypeStruct((B, ED), jnp.float32),
        mesh=mesh,
        scratch_types=[
            pltpu.VMEM((BPW,), jnp.int32),
            pltpu.VMEM((BPW, ED), jnp.float32),
            pltpu.SemaphoreType.DMA,
        ],
        compiler_params=pltpu.CompilerParams(needs_layout_passes=False, use_tc_tiling_on_sc=False),
    )
    def _gather_e(table_hbm, idx_hbm, out_hbm, idx_v, rows_v, sem):
        wid = lax.axis_index("s") * NC + lax.axis_index("c")
        base = wid * BPW
        pltpu.sync_copy(idx_hbm.at[pl.ds(base, BPW)], idx_v)
        pltpu.async_copy(table_hbm.at[idx_v], rows_v, sem).wait()
        pltpu.sync_copy(rows_v, out_hbm.at[pl.ds(base, BPW)])

    return _gather_e


def _mlp_body(e_ref, h_ref, w1_ref, b1_ref, w2_ref, b2_ref, relT_ref,
              x2e_ref, rel_ref):
    dot = functools.partial(
        jax.lax.dot_general,
        dimension_numbers=(((1,), (0,)), ((), ())),
        preferred_element_type=jnp.float32,
        precision=jax.lax.Precision.HIGHEST,
    )
    x = dot(e_ref[...], w1_ref[:ED, :]) + dot(h_ref[...], w1_ref[ED:, :])
    x = jnp.maximum(x + b1_ref[...], 0.0)
    x2 = jnp.maximum(dot(x, w2_ref[...]) + b2_ref[...], 0.0)
    x2e_ref[...] = x2[:, ED:]
    rel_ref[...] = dot(x2[:, :ED], relT_ref[...])


def _mlp(E, H, W1, b1, W2, b2, relT):
    bs = 512
    grid = (B // bs,)
    return pl.pallas_call(
        _mlp_body,
        grid=grid,
        in_specs=[
            pl.BlockSpec((bs, ED), lambda i: (i, 0)),
            pl.BlockSpec((bs, HD), lambda i: (i, 0)),
            pl.BlockSpec((ED + HD, AD), lambda i: (0, 0)),
            pl.BlockSpec((1, AD), lambda i: (0, 0)),
            pl.BlockSpec((AD, AD), lambda i: (0, 0)),
            pl.BlockSpec((1, AD), lambda i: (0, 0)),
            pl.BlockSpec((ED, NR_PAD), lambda i: (0, 0)),
        ],
        out_specs=[
            pl.BlockSpec((bs, ED), lambda i: (i, 0)),
            pl.BlockSpec((bs, NR_PAD), lambda i: (i, 0)),
        ],
        out_shape=[
            jax.ShapeDtypeStruct((B, ED), jnp.float32),
            jax.ShapeDtypeStruct((B, NR_PAD), jnp.float32),
        ],
    )(E, H, W1, b1, W2, b2, relT)


@functools.cache
def _build_scores_sc():
    mesh = plsc.VectorSubcoreMesh(core_axis_name="c", subcore_axis_name="s",
                                  num_cores=NC, num_subcores=NS)

    @functools.partial(
        pl.kernel,
        out_type=jax.ShapeDtypeStruct((B, A_OUT), jnp.float32),
        mesh=mesh,
        scratch_types=[
            pltpu.VMEM((2, CH, A_PAD), jnp.int32),
            pltpu.VMEM((2, CH, A_PAD), jnp.int32),
            pltpu.VMEM((2, CH, A_PAD), jnp.float32),
            pltpu.VMEM((BPW, ED), jnp.float32),
            pltpu.VMEM((2, CH, NR_PAD), jnp.float32),
            pltpu.VMEM((2, A_PAD, ED), jnp.float32),
            pltpu.VMEM((2, CH, A_OUT), jnp.float32),
            pltpu.SemaphoreType.DMA,
            pltpu.SemaphoreType.DMA,
            pltpu.SemaphoreType.DMA,
            pltpu.SemaphoreType.DMA,
            pltpu.SemaphoreType.DMA,
            pltpu.SemaphoreType.DMA,
        ],
        compiler_params=pltpu.CompilerParams(needs_layout_passes=False, use_tc_tiling_on_sc=False),
    )
    def _scores_sc(esp_hbm, rsp_hbm, msk_hbm, x2e_hbm, rel_hbm, table_hbm,
                   out_hbm, eidx_v, ridx_v, msk_v, x2e_v, rel_v, rows_v, sc_v,
                   csem0, csem1, gsem0, gsem1, osem0, osem1):
        _scores_body(esp_hbm, rsp_hbm, msk_hbm, x2e_hbm, rel_hbm, table_hbm,
                     out_hbm, eidx_v, ridx_v, msk_v, x2e_v, rel_v, rows_v,
                     sc_v, (csem0, csem1), (gsem0, gsem1), (osem0, osem1))

    return _scores_sc


def _scores_body(esp_hbm, rsp_hbm, msk_hbm, x2e_hbm, rel_hbm, table_hbm,
                 out_hbm, eidx_v, ridx_v, msk_v, x2e_v, rel_v, rows_v, sc_v,
                 csems, gsems, osems):
    wid = lax.axis_index("s") * NC + lax.axis_index("c")
    base = wid * BPW
    lane = lax.iota(jnp.int32, L)
    perms = tuple(lane ^ sh for sh in (8, 4, 2, 1))

    zi = jnp.zeros((L,), jnp.int32)
    zf = jnp.zeros((L,), jnp.float32)
    neg = jnp.full((L,), -1e30, jnp.float32)
    pad_idx = wid * L + lane
    for slot in range(2):
        for bb in range(CH):
            eidx_v[slot, bb, pl.ds(192, L)] = pad_idx
            ridx_v[slot, bb, pl.ds(192, L)] = zi
            msk_v[slot, bb, pl.ds(192, L)] = zf
            for g in (13, 14, 15):
                sc_v[slot, bb, pl.ds(g * L, L)] = neg

    def issue_chunk(slot, c):
        cb = base + c * CH
        for bb in range(CH):
            pltpu.async_copy(esp_hbm.at[cb + bb],
                             eidx_v.at[slot, bb, pl.ds(0, A)], csems[slot])
            pltpu.async_copy(rsp_hbm.at[cb + bb],
                             ridx_v.at[slot, bb, pl.ds(0, A)], csems[slot])
            pltpu.async_copy(msk_hbm.at[cb + bb],
                             msk_v.at[slot, bb, pl.ds(0, A)], csems[slot])
        pltpu.async_copy(rel_hbm.at[pl.ds(cb, CH)], rel_v.at[slot],
                         csems[slot])

    def wait_chunk(slot):
        for bb in range(CH):
            pltpu.make_async_copy(esp_hbm.at[base],
                                  eidx_v.at[slot, bb, pl.ds(0, A)],
                                  csems[slot]).wait()
            pltpu.make_async_copy(rsp_hbm.at[base],
                                  ridx_v.at[slot, bb, pl.ds(0, A)],
                                  csems[slot]).wait()
            pltpu.make_async_copy(msk_hbm.at[base],
                                  msk_v.at[slot, bb, pl.ds(0, A)],
                                  csems[slot]).wait()
        pltpu.make_async_copy(rel_hbm.at[pl.ds(base, CH)], rel_v.at[slot],
                              csems[slot]).wait()

    QPARTS = ((0, 32), (32, 24), (56, 24), (80, 24),
              (104, 32), (136, 24), (160, 24), (184, 24))

    def issue_gather(islot, brow, bslot):
        for off, n in QPARTS:
            pltpu.async_copy(table_hbm.at[eidx_v.at[islot, brow, pl.ds(off, n)]],
                             rows_v.at[bslot, pl.ds(off, n)], gsems[bslot])

    def wait_gather(bslot):
        for off, n in QPARTS:
            pltpu.make_async_copy(table_hbm.at[pl.ds(0, n)],
                                  rows_v.at[bslot, pl.ds(off, n)],
                                  gsems[bslot]).wait()

    def issue_out(slot, c):
        pltpu.async_copy(sc_v.at[slot], out_hbm.at[pl.ds(base + c * CH, CH)],
                         osems[slot])

    def wait_out(slot):
        pltpu.make_async_copy(sc_v.at[slot], out_hbm.at[pl.ds(base, CH)],
                              osems[slot]).wait()

    def _lane_sum(v):
        for p in perms:
            v = v + v.at[p].get(mode="promise_in_bounds")
        return v

    def compute(p, q, bb, b_local):
        xk = tuple(x2e_v[b_local, pl.ds(k * L, L)] for k in range(ED // L))

        def group_body(g, c):
            a0 = g * L
            res = jnp.zeros((L,), jnp.float32)
            for j in range(L):
                a = a0 + j
                acc = rows_v[q, a, pl.ds(0, L)] * xk[0]
                for k in range(1, ED // L):
                    acc = acc + rows_v[q, a, pl.ds(k * L, L)] * xk[k]
                res = jnp.where(lane == j, _lane_sum(acc), res)
            ri = ridx_v[p, bb, pl.ds(a0, L)]
            rv = plsc.load_gather(rel_v.at[p],
                                  [jnp.full((L,), bb, jnp.int32), ri])
            mv = msk_v[p, bb, pl.ds(a0, L)]
            sc_v[p, bb, pl.ds(a0, L)] = res + rv - (1.0 - mv) * HUGE
            return c

        lax.fori_loop(0, A_PAD // L, group_body, 0)

    pltpu.sync_copy(x2e_hbm.at[pl.ds(base, BPW)], x2e_v)
    issue_chunk(0, 0)
    wait_chunk(0)
    issue_gather(0, 0, 0)

    @pl.loop(0, NCHUNK, step=2)
    def chunk_loop(ci):
        for p in range(2):
            c = ci + p
            cnext = jnp.minimum(c + 1, NCHUNK - 1)

            @pl.when(c >= 2)
            def _():
                wait_out(p)

            issue_chunk(p ^ 1, cnext)

            @pl.loop(0, CH, step=2)
            def b_loop(qi):
                for q in range(2):
                    bb = qi + q
                    wait_gather(q)

                    @pl.when(bb < CH - 1)
                    def _():
                        issue_gather(p, bb + 1, q ^ 1)

                    @pl.when(bb == CH - 1)
                    def _():
                        wait_chunk(p ^ 1)
                        issue_gather(p ^ 1, 0, q ^ 1)

                    compute(p, q, bb, c * CH + bb)

            issue_out(p, c)

    wait_gather(0)
    wait_out(0)
    wait_out(1)


def _soft_body(s_ref, dist_ref, ent_ref):
    s = s_ref[...]
    m = jnp.max(s, axis=1, keepdims=True)
    ex = jnp.exp(s - m)
    z = jnp.sum(ex, axis=1, keepdims=True)
    p = ex / z
    dist_ref[...] = p
    ent_ref[...] = -jnp.sum(p * jnp.log(p + 1e-20), axis=1, keepdims=True)


def _softmax(scores):
    bs = 512
    return pl.pallas_call(
        _soft_body,
        grid=(B // bs,),
        in_specs=[pl.BlockSpec((bs, A_OUT), lambda i: (i, 0))],
        out_specs=[
            pl.BlockSpec((bs, A_OUT), lambda i: (i, 0)),
            pl.BlockSpec((bs, 1), lambda i: (i, 0)),
        ],
        out_shape=[
            jax.ShapeDtypeStruct((B, A_OUT), jnp.float32),
            jax.ShapeDtypeStruct((B, 1), jnp.float32),
        ],
    )(scores)


def kernel(e, H, r_space, e_space, action_mask, entity_emb, relation_emb,
           W1, b1, W2, b2):
    e = e.astype(jnp.int32)
    r_space = r_space.astype(jnp.int32)
    e_space = e_space.astype(jnp.int32)
    nr1 = relation_emb.shape[0]
    relT = jnp.zeros((ED, NR_PAD), jnp.float32).at[:, :nr1].set(relation_emb.T)

    E = _build_gather_e()(entity_emb, e)
    x2e, rel_scores = _mlp(E, H, W1, b1.reshape(1, AD), W2, b2.reshape(1, AD),
                           relT)
    scores = _build_scores_sc()(e_space, r_space, action_mask, x2e,
                                rel_scores, entity_emb)
    dist, ent = _softmax(scores)
    return dist[:, :A], ent.reshape(B)


# speedup vs baseline: 2.2389x; 1.0100x over previous
import functools

import jax
import jax.numpy as jnp
from jax import lax
from jax.experimental import pallas as pl
from jax.experimental.pallas import tpu as pltpu
from jax.experimental.pallas import tpu_sc as plsc

B = 4096
A = 200
A_PAD = 208
A_OUT = 256
ED = 128
HD = 256
AD = ED + HD // 2
NR_PAD = 512
HUGE = 1e9

NC, NS, L = 2, 16, 16
NW = NC * NS
BPW = B // NW
CH = 8
NCHUNK = BPW // CH
GH = A_PAD // 2

@functools.cache
def _build_gather_e():
    mesh = plsc.VectorSubcoreMesh(core_axis_name="c", subcore_axis_name="s",
                                  num_cores=NC, num_subcores=NS)

    @functools.partial(
        pl.kernel,
        out_type=jax.ShapeDtypeStruct((B, ED), jnp.float32),
        mesh=mesh,
        scratch_types=[
            pltpu.VMEM((BPW,), jnp.int32),
            pltpu.VMEM((BPW, ED), jnp.float32),
            pltpu.SemaphoreType.DMA,
        ],
        compiler_params=pltpu.CompilerParams(needs_layout_passes=False, use_tc_tiling_on_sc=False),
    )
    def _gather_e(table_hbm, idx_hbm, out_hbm, idx_v, rows_v, sem):
        wid = lax.axis_index("s") * NC + lax.axis_index("c")
        base = wid * BPW
        pltpu.sync_copy(idx_hbm.at[pl.ds(base, BPW)], idx_v)
        pltpu.async_copy(table_hbm.at[idx_v], rows_v, sem).wait()
        pltpu.sync_copy(rows_v, out_hbm.at[pl.ds(base, BPW)])

    return _gather_e


def _mlp_body(e_ref, h_ref, w1_ref, b1_ref, w2_ref, b2_ref, relT_ref,
              x2e_ref, rel_ref):
    dot = functools.partial(
        jax.lax.dot_general,
        dimension_numbers=(((1,), (0,)), ((), ())),
        preferred_element_type=jnp.float32,
        precision=jax.lax.Precision.HIGHEST,
    )
    x = dot(e_ref[...], w1_ref[:ED, :]) + dot(h_ref[...], w1_ref[ED:, :])
    x = jnp.maximum(x + b1_ref[...], 0.0)
    x2 = jnp.maximum(dot(x, w2_ref[...]) + b2_ref[...], 0.0)
    x2e_ref[...] = x2[:, ED:]
    rel_ref[...] = jax.lax.dot_general(
        x2[:, :ED], relT_ref[...],
        dimension_numbers=(((1,), (1,)), ((), ())),
        preferred_element_type=jnp.float32,
        precision=jax.lax.Precision.HIGHEST)


def _mlp(E, H, W1, b1, W2, b2, relT):
    bs = 512
    grid = (B // bs,)
    return pl.pallas_call(
        _mlp_body,
        grid=grid,
        in_specs=[
            pl.BlockSpec((bs, ED), lambda i: (i, 0)),
            pl.BlockSpec((bs, HD), lambda i: (i, 0)),
            pl.BlockSpec((ED + HD, AD), lambda i: (0, 0)),
            pl.BlockSpec((1, AD), lambda i: (0, 0)),
            pl.BlockSpec((AD, AD), lambda i: (0, 0)),
            pl.BlockSpec((1, AD), lambda i: (0, 0)),
            pl.BlockSpec((NR_PAD, ED), lambda i: (0, 0)),
        ],
        out_specs=[
            pl.BlockSpec((bs, ED), lambda i: (i, 0)),
            pl.BlockSpec((bs, NR_PAD), lambda i: (i, 0)),
        ],
        out_shape=[
            jax.ShapeDtypeStruct((B, ED), jnp.float32),
            jax.ShapeDtypeStruct((B, NR_PAD), jnp.float32),
        ],
    )(E, H, W1, b1, W2, b2, relT)


@functools.cache
def _build_scores_sc():
    mesh = plsc.VectorSubcoreMesh(core_axis_name="c", subcore_axis_name="s",
                                  num_cores=NC, num_subcores=NS)

    @functools.partial(
        pl.kernel,
        out_type=jax.ShapeDtypeStruct((B, A_OUT), jnp.float32),
        mesh=mesh,
        scratch_types=[
            pltpu.VMEM((2, CH, A_PAD), jnp.int32),
            pltpu.VMEM((2, CH, A_PAD), jnp.int32),
            pltpu.VMEM((2, CH, A_PAD), jnp.float32),
            pltpu.VMEM((BPW, ED), jnp.float32),
            pltpu.VMEM((2, CH, NR_PAD), jnp.float32),
            pltpu.VMEM((2, A_PAD, ED), jnp.float32),
            pltpu.VMEM((2, CH, A_OUT), jnp.float32),
            pltpu.SemaphoreType.DMA,
            pltpu.SemaphoreType.DMA,
            pltpu.SemaphoreType.DMA,
            pltpu.SemaphoreType.DMA,
            pltpu.SemaphoreType.DMA,
            pltpu.SemaphoreType.DMA,
        ],
        compiler_params=pltpu.CompilerParams(needs_layout_passes=False, use_tc_tiling_on_sc=False),
    )
    def _scores_sc(esp_hbm, rsp_hbm, msk_hbm, x2e_hbm, rel_hbm, table_hbm,
                   out_hbm, eidx_v, ridx_v, msk_v, x2e_v, rel_v, rows_v, sc_v,
                   csem0, csem1, gsem0, gsem1, osem0, osem1):
        _scores_body(esp_hbm, rsp_hbm, msk_hbm, x2e_hbm, rel_hbm, table_hbm,
                     out_hbm, eidx_v, ridx_v, msk_v, x2e_v, rel_v, rows_v,
                     sc_v, (csem0, csem1), (gsem0, gsem1), (osem0, osem1))

    return _scores_sc


def _scores_body(esp_hbm, rsp_hbm, msk_hbm, x2e_hbm, rel_hbm, table_hbm,
                 out_hbm, eidx_v, ridx_v, msk_v, x2e_v, rel_v, rows_v, sc_v,
                 csems, gsems, osems):
    wid = lax.axis_index("s") * NC + lax.axis_index("c")
    base = wid * BPW
    lane = lax.iota(jnp.int32, L)
    perms = tuple(lane ^ sh for sh in (8, 4, 2, 1))

    zi = jnp.zeros((L,), jnp.int32)
    zf = jnp.zeros((L,), jnp.float32)
    neg = jnp.full((L,), -1e30, jnp.float32)
    for slot in range(2):
        for bb in range(CH):
            ridx_v[slot, bb, pl.ds(192, L)] = zi
            msk_v[slot, bb, pl.ds(192, L)] = zf
            for g in (13, 14, 15):
                sc_v[slot, bb, pl.ds(g * L, L)] = neg

    def issue_chunk(slot, c):
        cb = base + c * CH
        for bb in range(CH):
            pltpu.async_copy(esp_hbm.at[cb + bb],
                             eidx_v.at[slot, bb, pl.ds(0, A)], csems[slot])
            pltpu.async_copy(rsp_hbm.at[cb + bb],
                             ridx_v.at[slot, bb, pl.ds(0, A)], csems[slot])
            pltpu.async_copy(msk_hbm.at[cb + bb],
                             msk_v.at[slot, bb, pl.ds(0, A)], csems[slot])
        pltpu.async_copy(rel_hbm.at[pl.ds(cb, CH)], rel_v.at[slot],
                         csems[slot])

    def wait_chunk(slot):
        for bb in range(CH):
            pltpu.make_async_copy(esp_hbm.at[base],
                                  eidx_v.at[slot, bb, pl.ds(0, A)],
                                  csems[slot]).wait()
            pltpu.make_async_copy(rsp_hbm.at[base],
                                  ridx_v.at[slot, bb, pl.ds(0, A)],
                                  csems[slot]).wait()
            pltpu.make_async_copy(msk_hbm.at[base],
                                  msk_v.at[slot, bb, pl.ds(0, A)],
                                  csems[slot]).wait()
        pltpu.make_async_copy(rel_hbm.at[pl.ds(base, CH)], rel_v.at[slot],
                              csems[slot]).wait()

    QPARTS = ((0, 56), (56, 48), (104, 56), (160, 40))

    def issue_gather(islot, brow, bslot):
        for off, n in QPARTS:
            pltpu.async_copy(table_hbm.at[eidx_v.at[islot, brow, pl.ds(off, n)]],
                             rows_v.at[bslot, pl.ds(off, n)], gsems[bslot])

    def wait_gather(bslot):
        for off, n in QPARTS:
            pltpu.make_async_copy(table_hbm.at[pl.ds(0, n)],
                                  rows_v.at[bslot, pl.ds(off, n)],
                                  gsems[bslot]).wait()

    def issue_out(slot, c):
        pltpu.async_copy(sc_v.at[slot], out_hbm.at[pl.ds(base + c * CH, CH)],
                         osems[slot])

    def wait_out(slot):
        pltpu.make_async_copy(sc_v.at[slot], out_hbm.at[pl.ds(base, CH)],
                              osems[slot]).wait()

    def _lane_sum(v):
        for p in perms:
            v = v + v.at[p].get(mode="promise_in_bounds")
        return v

    def compute(p, q, bb, b_local):
        xk = tuple(x2e_v[b_local, pl.ds(k * L, L)] for k in range(ED // L))

        def finish_group(a0, res):
            ri = ridx_v[p, bb, pl.ds(a0, L)]
            rv = plsc.load_gather(rel_v.at[p],
                                  [jnp.full((L,), bb, jnp.int32), ri])
            mv = msk_v[p, bb, pl.ds(a0, L)]
            sc_v[p, bb, pl.ds(a0, L)] = res + rv - (1.0 - mv) * HUGE

        def dots(a0, njs):
            res = jnp.zeros((L,), jnp.float32)
            for j in range(njs):
                a = a0 + j
                acc = rows_v[q, a, pl.ds(0, L)] * xk[0]
                for k in range(1, ED // L):
                    acc = acc + rows_v[q, a, pl.ds(k * L, L)] * xk[k]
                res = jnp.where(lane == j, _lane_sum(acc), res)
            return res

        def group_body(g, c):
            a0 = g * L
            finish_group(a0, dots(a0, L))
            return c

        lax.fori_loop(0, A // L, group_body, 0)
        finish_group(192, dots(192, A - (A // L) * L))

    pltpu.sync_copy(x2e_hbm.at[pl.ds(base, BPW)], x2e_v)
    issue_chunk(0, 0)
    wait_chunk(0)
    issue_gather(0, 0, 0)

    @pl.loop(0, NCHUNK, step=2)
    def chunk_loop(ci):
        for p in range(2):
            c = ci + p
            cnext = jnp.minimum(c + 1, NCHUNK - 1)

            @pl.when(c >= 2)
            def _():
                wait_out(p)

            issue_chunk(p ^ 1, cnext)

            @pl.loop(0, CH, step=2)
            def b_loop(qi):
                for q in range(2):
                    bb = qi + q
                    wait_gather(q)

                    @pl.when(bb < CH - 1)
                    def _():
                        issue_gather(p, bb + 1, q ^ 1)

                    @pl.when(bb == CH - 1)
                    def _():
                        wait_chunk(p ^ 1)
                        issue_gather(p ^ 1, 0, q ^ 1)

                    compute(p, q, bb, c * CH + bb)

            issue_out(p, c)

    wait_gather(0)
    wait_out(0)
    wait_out(1)


def _soft_body(s_ref, dist_ref, ent_ref):
    s = s_ref[...]
    m = jnp.max(s, axis=1, keepdims=True)
    ex = jnp.exp(s - m)
    z = jnp.sum(ex, axis=1, keepdims=True)
    p = ex / z
    dist_ref[...] = p[:, :A]
    ent_ref[...] = -jnp.sum(p * jnp.log(p + 1e-20), axis=1, keepdims=True)


def _softmax(scores):
    bs = 512
    return pl.pallas_call(
        _soft_body,
        grid=(B // bs,),
        in_specs=[pl.BlockSpec((bs, A_OUT), lambda i: (i, 0))],
        out_specs=[
            pl.BlockSpec((bs, A), lambda i: (i, 0)),
            pl.BlockSpec((bs, 1), lambda i: (i, 0)),
        ],
        out_shape=[
            jax.ShapeDtypeStruct((B, A), jnp.float32),
            jax.ShapeDtypeStruct((B, 1), jnp.float32),
        ],
    )(scores)


def kernel(e, H, r_space, e_space, action_mask, entity_emb, relation_emb,
           W1, b1, W2, b2):
    e = e.astype(jnp.int32)
    r_space = r_space.astype(jnp.int32)
    e_space = e_space.astype(jnp.int32)
    nr1 = relation_emb.shape[0]
    relP = jnp.pad(relation_emb, ((0, NR_PAD - nr1), (0, 0)))

    E = _build_gather_e()(entity_emb, e)
    x2e, rel_scores = _mlp(E, H, W1, b1.reshape(1, AD), W2, b2.reshape(1, AD),
                           relP)
    scores = _build_scores_sc()(e_space, r_space, action_mask, x2e,
                                rel_scores, entity_emb)
    dist, ent = _softmax(scores)
    return dist, ent.reshape(B)


# speedup vs baseline: 2.3571x; 1.0528x over previous
import functools

import jax
import jax.numpy as jnp
from jax import lax
from jax.experimental import pallas as pl
from jax.experimental.pallas import tpu as pltpu
from jax.experimental.pallas import tpu_sc as plsc

B = 4096
A = 200
A_PAD = 208
A_OUT = 256
ED = 128
HD = 256
AD = ED + HD // 2
NR_PAD = 512
HUGE = 1e9

NC, NS, L = 2, 16, 16
NW = NC * NS
BPW = B // NW
CH = 8
NCHUNK = BPW // CH
GH = A_PAD // 2

@functools.cache
def _build_gather_e():
    mesh = plsc.VectorSubcoreMesh(core_axis_name="c", subcore_axis_name="s",
                                  num_cores=NC, num_subcores=NS)

    @functools.partial(
        pl.kernel,
        out_type=jax.ShapeDtypeStruct((B, ED), jnp.float32),
        mesh=mesh,
        scratch_types=[
            pltpu.VMEM((BPW,), jnp.int32),
            pltpu.VMEM((BPW, ED), jnp.float32),
            pltpu.SemaphoreType.DMA,
        ],
        compiler_params=pltpu.CompilerParams(needs_layout_passes=False, use_tc_tiling_on_sc=False),
    )
    def _gather_e(table_hbm, idx_hbm, out_hbm, idx_v, rows_v, sem):
        wid = lax.axis_index("s") * NC + lax.axis_index("c")
        base = wid * BPW
        pltpu.sync_copy(idx_hbm.at[pl.ds(base, BPW)], idx_v)
        pltpu.async_copy(table_hbm.at[idx_v], rows_v, sem).wait()
        pltpu.sync_copy(rows_v, out_hbm.at[pl.ds(base, BPW)])

    return _gather_e


def _mlp_body(e_ref, h_ref, w1_ref, b1_ref, w2_ref, b2_ref, relT_ref,
              x2e_ref, rel_ref):
    dot = functools.partial(
        jax.lax.dot_general,
        dimension_numbers=(((1,), (0,)), ((), ())),
        preferred_element_type=jnp.float32,
    )
    x = dot(e_ref[...], w1_ref[:ED, :]) + dot(h_ref[...], w1_ref[ED:, :])
    x = jnp.maximum(x + b1_ref[...], 0.0)
    x2 = jnp.maximum(dot(x, w2_ref[...]) + b2_ref[...], 0.0)
    x2e_ref[...] = x2[:, ED:]
    rel_ref[...] = jax.lax.dot_general(
        x2[:, :ED], relT_ref[...],
        dimension_numbers=(((1,), (1,)), ((), ())),
        preferred_element_type=jnp.float32)


def _mlp(E, H, W1, b1, W2, b2, relT):
    bs = 512
    grid = (B // bs,)
    return pl.pallas_call(
        _mlp_body,
        grid=grid,
        in_specs=[
            pl.BlockSpec((bs, ED), lambda i: (i, 0)),
            pl.BlockSpec((bs, HD), lambda i: (i, 0)),
            pl.BlockSpec((ED + HD, AD), lambda i: (0, 0)),
            pl.BlockSpec((1, AD), lambda i: (0, 0)),
            pl.BlockSpec((AD, AD), lambda i: (0, 0)),
            pl.BlockSpec((1, AD), lambda i: (0, 0)),
            pl.BlockSpec((NR_PAD, ED), lambda i: (0, 0)),
        ],
        out_specs=[
            pl.BlockSpec((bs, ED), lambda i: (i, 0)),
            pl.BlockSpec((bs, NR_PAD), lambda i: (i, 0)),
        ],
        out_shape=[
            jax.ShapeDtypeStruct((B, ED), jnp.float32),
            jax.ShapeDtypeStruct((B, NR_PAD), jnp.float32),
        ],
    )(E, H, W1, b1, W2, b2, relT)


@functools.cache
def _build_scores_sc():
    mesh = plsc.VectorSubcoreMesh(core_axis_name="c", subcore_axis_name="s",
                                  num_cores=NC, num_subcores=NS)

    @functools.partial(
        pl.kernel,
        out_type=jax.ShapeDtypeStruct((B, A_OUT), jnp.float32),
        mesh=mesh,
        scratch_types=[
            pltpu.VMEM((2, CH, A_PAD), jnp.int32),
            pltpu.VMEM((2, CH, A_PAD), jnp.int32),
            pltpu.VMEM((2, CH, A_PAD), jnp.float32),
            pltpu.VMEM((BPW, ED), jnp.float32),
            pltpu.VMEM((2, CH, NR_PAD), jnp.float32),
            pltpu.VMEM((2, A_PAD, ED), jnp.float32),
            pltpu.VMEM((2, CH, A_OUT), jnp.float32),
            pltpu.SemaphoreType.DMA,
            pltpu.SemaphoreType.DMA,
            pltpu.SemaphoreType.DMA,
            pltpu.SemaphoreType.DMA,
            pltpu.SemaphoreType.DMA,
            pltpu.SemaphoreType.DMA,
        ],
        compiler_params=pltpu.CompilerParams(needs_layout_passes=False, use_tc_tiling_on_sc=False),
    )
    def _scores_sc(esp_hbm, rsp_hbm, msk_hbm, x2e_hbm, rel_hbm, table_hbm,
                   out_hbm, eidx_v, ridx_v, msk_v, x2e_v, rel_v, rows_v, sc_v,
                   csem0, csem1, gsem0, gsem1, osem0, osem1):
        _scores_body(esp_hbm, rsp_hbm, msk_hbm, x2e_hbm, rel_hbm, table_hbm,
                     out_hbm, eidx_v, ridx_v, msk_v, x2e_v, rel_v, rows_v,
                     sc_v, (csem0, csem1), (gsem0, gsem1), (osem0, osem1))

    return _scores_sc


def _scores_body(esp_hbm, rsp_hbm, msk_hbm, x2e_hbm, rel_hbm, table_hbm,
                 out_hbm, eidx_v, ridx_v, msk_v, x2e_v, rel_v, rows_v, sc_v,
                 csems, gsems, osems):
    wid = lax.axis_index("s") * NC + lax.axis_index("c")
    base = wid * BPW
    lane = lax.iota(jnp.int32, L)
    perms = tuple(lane ^ sh for sh in (8, 4, 2, 1))

    zi = jnp.zeros((L,), jnp.int32)
    zf = jnp.zeros((L,), jnp.float32)
    neg = jnp.full((L,), -1e30, jnp.float32)
    for slot in range(2):
        for bb in range(CH):
            ridx_v[slot, bb, pl.ds(192, L)] = zi
            msk_v[slot, bb, pl.ds(192, L)] = zf
            for g in (13, 14, 15):
                sc_v[slot, bb, pl.ds(g * L, L)] = neg

    def issue_chunk(slot, c):
        cb = base + c * CH
        for bb in range(CH):
            pltpu.async_copy(esp_hbm.at[cb + bb],
                             eidx_v.at[slot, bb, pl.ds(0, A)], csems[slot])
            pltpu.async_copy(rsp_hbm.at[cb + bb],
                             ridx_v.at[slot, bb, pl.ds(0, A)], csems[slot])
            pltpu.async_copy(msk_hbm.at[cb + bb],
                             msk_v.at[slot, bb, pl.ds(0, A)], csems[slot])
        pltpu.async_copy(rel_hbm.at[pl.ds(cb, CH)], rel_v.at[slot],
                         csems[slot])

    def wait_chunk(slot):
        for bb in range(CH):
            pltpu.make_async_copy(esp_hbm.at[base],
                                  eidx_v.at[slot, bb, pl.ds(0, A)],
                                  csems[slot]).wait()
            pltpu.make_async_copy(rsp_hbm.at[base],
                                  ridx_v.at[slot, bb, pl.ds(0, A)],
                                  csems[slot]).wait()
            pltpu.make_async_copy(msk_hbm.at[base],
                                  msk_v.at[slot, bb, pl.ds(0, A)],
                                  csems[slot]).wait()
        pltpu.make_async_copy(rel_hbm.at[pl.ds(base, CH)], rel_v.at[slot],
                              csems[slot]).wait()

    QPARTS = ((0, 56), (56, 48), (104, 56), (160, 40))

    def issue_gather(islot, brow, bslot):
        for off, n in QPARTS:
            pltpu.async_copy(table_hbm.at[eidx_v.at[islot, brow, pl.ds(off, n)]],
                             rows_v.at[bslot, pl.ds(off, n)], gsems[bslot])

    def wait_gather(bslot):
        for off, n in QPARTS:
            pltpu.make_async_copy(table_hbm.at[pl.ds(0, n)],
                                  rows_v.at[bslot, pl.ds(off, n)],
                                  gsems[bslot]).wait()

    def issue_out(slot, c):
        pltpu.async_copy(sc_v.at[slot], out_hbm.at[pl.ds(base + c * CH, CH)],
                         osems[slot])

    def wait_out(slot):
        pltpu.make_async_copy(sc_v.at[slot], out_hbm.at[pl.ds(base, CH)],
                              osems[slot]).wait()

    def _lane_sum(v):
        for p in perms:
            v = v + v.at[p].get(mode="promise_in_bounds")
        return v

    def compute(p, q, bb, b_local):
        xk = tuple(x2e_v[b_local, pl.ds(k * L, L)] for k in range(ED // L))

        def finish_group(a0, res):
            ri = ridx_v[p, bb, pl.ds(a0, L)]
            rv = plsc.load_gather(rel_v.at[p],
                                  [jnp.full((L,), bb, jnp.int32), ri])
            mv = msk_v[p, bb, pl.ds(a0, L)]
            sc_v[p, bb, pl.ds(a0, L)] = res + rv - (1.0 - mv) * HUGE

        def dots(a0, njs):
            res = jnp.zeros((L,), jnp.float32)
            for j in range(njs):
                a = a0 + j
                acc = rows_v[q, a, pl.ds(0, L)] * xk[0]
                for k in range(1, ED // L):
                    acc = acc + rows_v[q, a, pl.ds(k * L, L)] * xk[k]
                res = jnp.where(lane == j, _lane_sum(acc), res)
            return res

        def group_body(g, c):
            a0 = g * L
            finish_group(a0, dots(a0, L))
            return c

        lax.fori_loop(0, A // L, group_body, 0)
        finish_group(192, dots(192, A - (A // L) * L))

    pltpu.sync_copy(x2e_hbm.at[pl.ds(base, BPW)], x2e_v)
    issue_chunk(0, 0)
    wait_chunk(0)
    issue_gather(0, 0, 0)

    @pl.loop(0, NCHUNK, step=2)
    def chunk_loop(ci):
        for p in range(2):
            c = ci + p
            cnext = jnp.minimum(c + 1, NCHUNK - 1)

            @pl.when(c >= 2)
            def _():
                wait_out(p)

            issue_chunk(p ^ 1, cnext)

            @pl.loop(0, CH, step=2)
            def b_loop(qi):
                for q in range(2):
                    bb = qi + q
                    wait_gather(q)

                    @pl.when(bb < CH - 1)
                    def _():
                        issue_gather(p, bb + 1, q ^ 1)

                    @pl.when(bb == CH - 1)
                    def _():
                        wait_chunk(p ^ 1)
                        issue_gather(p ^ 1, 0, q ^ 1)

                    compute(p, q, bb, c * CH + bb)

            issue_out(p, c)

    wait_gather(0)
    wait_out(0)
    wait_out(1)


def _soft_body(s_ref, dist_ref, ent_ref):
    s = s_ref[...]
    m = jnp.max(s, axis=1, keepdims=True)
    ex = jnp.exp(s - m)
    z = jnp.sum(ex, axis=1, keepdims=True)
    p = ex / z
    dist_ref[...] = p[:, :A]
    ent_ref[...] = -jnp.sum(p * jnp.log(p + 1e-20), axis=1, keepdims=True)


def _softmax(scores):
    bs = 512
    return pl.pallas_call(
        _soft_body,
        grid=(B // bs,),
        in_specs=[pl.BlockSpec((bs, A_OUT), lambda i: (i, 0))],
        out_specs=[
            pl.BlockSpec((bs, A), lambda i: (i, 0)),
            pl.BlockSpec((bs, 1), lambda i: (i, 0)),
        ],
        out_shape=[
            jax.ShapeDtypeStruct((B, A), jnp.float32),
            jax.ShapeDtypeStruct((B, 1), jnp.float32),
        ],
    )(scores)


def kernel(e, H, r_space, e_space, action_mask, entity_emb, relation_emb,
           W1, b1, W2, b2):
    e = e.astype(jnp.int32)
    r_space = r_space.astype(jnp.int32)
    e_space = e_space.astype(jnp.int32)
    nr1 = relation_emb.shape[0]
    relP = jnp.pad(relation_emb, ((0, NR_PAD - nr1), (0, 0)))

    E = _build_gather_e()(entity_emb, e)
    x2e, rel_scores = _mlp(E, H, W1, b1.reshape(1, AD), W2, b2.reshape(1, AD),
                           relP)
    scores = _build_scores_sc()(e_space, r_space, action_mask, x2e,
                                rel_scores, entity_emb)
    dist, ent = _softmax(scores)
    return dist, ent.reshape(B)


# speedup vs baseline: 2.3650x; 1.0034x over previous
import functools

import jax
import jax.numpy as jnp
from jax import lax
from jax.experimental import pallas as pl
from jax.experimental.pallas import tpu as pltpu
from jax.experimental.pallas import tpu_sc as plsc

B = 4096
A = 200
A_PAD = 208
A_OUT = 256
ED = 128
HD = 256
AD = ED + HD // 2
NR_PAD = 512
HUGE = 1e9

NC, NS, L = 2, 16, 16
NW = NC * NS
BPW = B // NW
CH = 8
NCHUNK = BPW // CH

@functools.cache
def _build_gather_e():
    mesh = plsc.VectorSubcoreMesh(core_axis_name="c", subcore_axis_name="s",
                                  num_cores=NC, num_subcores=NS)

    @functools.partial(
        pl.kernel,
        out_type=jax.ShapeDtypeStruct((B, ED), jnp.float32),
        mesh=mesh,
        scratch_types=[
            pltpu.VMEM((BPW,), jnp.int32),
            pltpu.VMEM((BPW, ED), jnp.float32),
            pltpu.SemaphoreType.DMA,
        ],
        compiler_params=pltpu.CompilerParams(needs_layout_passes=False, use_tc_tiling_on_sc=False),
    )
    def _gather_e(table_hbm, idx_hbm, out_hbm, idx_v, rows_v, sem):
        wid = lax.axis_index("s") * NC + lax.axis_index("c")
        base = wid * BPW
        pltpu.sync_copy(idx_hbm.at[pl.ds(base, BPW)], idx_v)
        pltpu.async_copy(table_hbm.at[idx_v], rows_v, sem).wait()
        pltpu.sync_copy(rows_v, out_hbm.at[pl.ds(base, BPW)])

    return _gather_e


def _mlp_body(e_ref, h_ref, w1_ref, b1_ref, w2_ref, b2_ref, relT_ref,
              x2e_ref, rel_ref):
    dot = functools.partial(
        jax.lax.dot_general,
        dimension_numbers=(((1,), (0,)), ((), ())),
        preferred_element_type=jnp.float32,
    )
    x = dot(e_ref[...], w1_ref[:ED, :]) + dot(h_ref[...], w1_ref[ED:, :])
    x = jnp.maximum(x + b1_ref[...], 0.0)
    x2 = jnp.maximum(dot(x, w2_ref[...]) + b2_ref[...], 0.0)
    x2e_ref[...] = x2[:, ED:]
    rel_ref[...] = jax.lax.dot_general(
        x2[:, :ED], relT_ref[...],
        dimension_numbers=(((1,), (1,)), ((), ())),
        preferred_element_type=jnp.float32)


def _mlp(E, H, W1, b1, W2, b2, relT):
    bs = 512
    grid = (B // bs,)
    return pl.pallas_call(
        _mlp_body,
        grid=grid,
        in_specs=[
            pl.BlockSpec((bs, ED), lambda i: (i, 0)),
            pl.BlockSpec((bs, HD), lambda i: (i, 0)),
            pl.BlockSpec((ED + HD, AD), lambda i: (0, 0)),
            pl.BlockSpec((1, AD), lambda i: (0, 0)),
            pl.BlockSpec((AD, AD), lambda i: (0, 0)),
            pl.BlockSpec((1, AD), lambda i: (0, 0)),
            pl.BlockSpec((NR_PAD, ED), lambda i: (0, 0)),
        ],
        out_specs=[
            pl.BlockSpec((bs, ED), lambda i: (i, 0)),
            pl.BlockSpec((bs, NR_PAD), lambda i: (i, 0)),
        ],
        out_shape=[
            jax.ShapeDtypeStruct((B, ED), jnp.float32),
            jax.ShapeDtypeStruct((B, NR_PAD), jnp.float32),
        ],
    )(E, H, W1, b1, W2, b2, relT)


@functools.cache
def _build_scores_sc():
    mesh = plsc.VectorSubcoreMesh(core_axis_name="c", subcore_axis_name="s",
                                  num_cores=NC, num_subcores=NS)

    @functools.partial(
        pl.kernel,
        out_type=jax.ShapeDtypeStruct((B, A_OUT), jnp.float32),
        mesh=mesh,
        scratch_types=[
            pltpu.VMEM((2, CH, A_PAD), jnp.int32),
            pltpu.VMEM((2, CH, A_PAD), jnp.int32),
            pltpu.VMEM((2, CH, A_PAD), jnp.float32),
            pltpu.VMEM((BPW, ED), jnp.float32),
            pltpu.VMEM((2, CH, NR_PAD), jnp.float32),
            pltpu.VMEM((2, A_PAD, ED), jnp.float32),
            pltpu.VMEM((2, CH, A_OUT), jnp.float32),
            pltpu.SemaphoreType.DMA,
            pltpu.SemaphoreType.DMA,
            pltpu.SemaphoreType.DMA,
            pltpu.SemaphoreType.DMA,
            pltpu.SemaphoreType.DMA,
            pltpu.SemaphoreType.DMA,
        ],
        compiler_params=pltpu.CompilerParams(needs_layout_passes=False, use_tc_tiling_on_sc=False),
    )
    def _scores_sc(esp_hbm, rsp_hbm, msk_hbm, x2e_hbm, rel_hbm, table_hbm,
                   out_hbm, eidx_v, ridx_v, msk_v, x2e_v, rel_v, rows_v, sc_v,
                   csem0, csem1, gsem0, gsem1, osem0, osem1):
        _scores_body(esp_hbm, rsp_hbm, msk_hbm, x2e_hbm, rel_hbm, table_hbm,
                     out_hbm, eidx_v, ridx_v, msk_v, x2e_v, rel_v, rows_v,
                     sc_v, (csem0, csem1), (gsem0, gsem1), (osem0, osem1))

    return _scores_sc


def _scores_body(esp_hbm, rsp_hbm, msk_hbm, x2e_hbm, rel_hbm, table_hbm,
                 out_hbm, eidx_v, ridx_v, msk_v, x2e_v, rel_v, rows_v, sc_v,
                 csems, gsems, osems):
    wid = lax.axis_index("s") * NC + lax.axis_index("c")
    base = wid * BPW
    lane = lax.iota(jnp.int32, L)
    perms = tuple(lane ^ sh for sh in (8, 4, 2, 1))

    zi = jnp.zeros((L,), jnp.int32)
    zf = jnp.zeros((L,), jnp.float32)
    neg = jnp.full((L,), -1e30, jnp.float32)
    for slot in range(2):
        for bb in range(CH):
            ridx_v[slot, bb, pl.ds(192, L)] = zi
            msk_v[slot, bb, pl.ds(192, L)] = zf
            for g in (13, 14, 15):
                sc_v[slot, bb, pl.ds(g * L, L)] = neg

    def issue_chunk(slot, c):
        cb = base + c * CH
        for bb in range(CH):
            pltpu.async_copy(esp_hbm.at[cb + bb],
                             eidx_v.at[slot, bb, pl.ds(0, A)], csems[slot])
            pltpu.async_copy(rsp_hbm.at[cb + bb],
                             ridx_v.at[slot, bb, pl.ds(0, A)], csems[slot])
            pltpu.async_copy(msk_hbm.at[cb + bb],
                             msk_v.at[slot, bb, pl.ds(0, A)], csems[slot])
        pltpu.async_copy(rel_hbm.at[pl.ds(cb, CH)], rel_v.at[slot],
                         csems[slot])

    def wait_chunk(slot):
        for bb in range(CH):
            pltpu.make_async_copy(esp_hbm.at[base],
                                  eidx_v.at[slot, bb, pl.ds(0, A)],
                                  csems[slot]).wait()
            pltpu.make_async_copy(rsp_hbm.at[base],
                                  ridx_v.at[slot, bb, pl.ds(0, A)],
                                  csems[slot]).wait()
            pltpu.make_async_copy(msk_hbm.at[base],
                                  msk_v.at[slot, bb, pl.ds(0, A)],
                                  csems[slot]).wait()
        pltpu.make_async_copy(rel_hbm.at[pl.ds(base, CH)], rel_v.at[slot],
                              csems[slot]).wait()

    QPARTS = ((0, 56), (56, 48), (104, 56), (160, 40))

    def issue_gather(islot, brow, bslot):
        for off, n in QPARTS:
            pltpu.async_copy(table_hbm.at[eidx_v.at[islot, brow, pl.ds(off, n)]],
                             rows_v.at[bslot, pl.ds(off, n)], gsems[bslot])

    def wait_gather(bslot):
        for off, n in QPARTS:
            pltpu.make_async_copy(table_hbm.at[pl.ds(0, n)],
                                  rows_v.at[bslot, pl.ds(off, n)],
                                  gsems[bslot]).wait()

    def issue_out(slot, c):
        pltpu.async_copy(sc_v.at[slot], out_hbm.at[pl.ds(base + c * CH, CH)],
                         osems[slot])

    def wait_out(slot):
        pltpu.make_async_copy(sc_v.at[slot], out_hbm.at[pl.ds(base, CH)],
                              osems[slot]).wait()

    def _lane_sum(v):
        for p in perms:
            v = v + v.at[p].get(mode="promise_in_bounds")
        return v

    def compute(p, q, bb, b_local):
        xk = tuple(x2e_v[b_local, pl.ds(k * L, L)] for k in range(ED // L))

        def finish_group(a0, res):
            ri = ridx_v[p, bb, pl.ds(a0, L)]
            rv = plsc.load_gather(rel_v.at[p],
                                  [jnp.full((L,), bb, jnp.int32), ri])
            mv = msk_v[p, bb, pl.ds(a0, L)]
            sc_v[p, bb, pl.ds(a0, L)] = res + rv - (1.0 - mv) * HUGE

        def dots(a0, njs):
            res = jnp.zeros((L,), jnp.float32)
            for j in range(njs):
                a = a0 + j
                acc = rows_v[q, a, pl.ds(0, L)] * xk[0]
                for k in range(1, ED // L):
                    acc = acc + rows_v[q, a, pl.ds(k * L, L)] * xk[k]
                res = jnp.where(lane == j, _lane_sum(acc), res)
            return res

        def group_body(g, c):
            a0 = g * L
            finish_group(a0, dots(a0, L))
            return c

        lax.fori_loop(0, A // L, group_body, 0)
        finish_group(192, dots(192, A - (A // L) * L))

    xcp = pltpu.async_copy(x2e_hbm.at[pl.ds(base, BPW)], x2e_v, osems[0])
    issue_chunk(0, 0)
    wait_chunk(0)
    issue_gather(0, 0, 0)
    xcp.wait()

    @pl.loop(0, NCHUNK, step=2)
    def chunk_loop(ci):
        for p in range(2):
            c = ci + p
            cnext = jnp.minimum(c + 1, NCHUNK - 1)

            @pl.when(c >= 2)
            def _():
                wait_out(p)

            issue_chunk(p ^ 1, cnext)

            @pl.loop(0, CH, step=2)
            def b_loop(qi):
                for q in range(2):
                    bb = qi + q
                    wait_gather(q)

                    @pl.when(bb < CH - 1)
                    def _():
                        issue_gather(p, bb + 1, q ^ 1)

                    @pl.when(bb == CH - 1)
                    def _():
                        wait_chunk(p ^ 1)
                        issue_gather(p ^ 1, 0, q ^ 1)

                    compute(p, q, bb, c * CH + bb)

            issue_out(p, c)

    wait_gather(0)
    wait_out(0)
    wait_out(1)


def _soft_body(s_ref, dist_ref, ent_ref):
    s = s_ref[...]
    m = jnp.max(s, axis=1, keepdims=True)
    ex = jnp.exp(s - m)
    z = jnp.sum(ex, axis=1, keepdims=True)
    p = ex / z
    dist_ref[...] = p[:, :A]
    ent_ref[...] = -jnp.sum(p * jnp.log(p + 1e-20), axis=1, keepdims=True)


def _softmax(scores):
    bs = 512
    return pl.pallas_call(
        _soft_body,
        grid=(B // bs,),
        in_specs=[pl.BlockSpec((bs, A_OUT), lambda i: (i, 0))],
        out_specs=[
            pl.BlockSpec((bs, A), lambda i: (i, 0)),
            pl.BlockSpec((bs, 1), lambda i: (i, 0)),
        ],
        out_shape=[
            jax.ShapeDtypeStruct((B, A), jnp.float32),
            jax.ShapeDtypeStruct((B, 1), jnp.float32),
        ],
    )(scores)


def kernel(e, H, r_space, e_space, action_mask, entity_emb, relation_emb,
           W1, b1, W2, b2):
    e = e.astype(jnp.int32)
    r_space = r_space.astype(jnp.int32)
    e_space = e_space.astype(jnp.int32)
    nr1 = relation_emb.shape[0]
    relP = jnp.pad(relation_emb, ((0, NR_PAD - nr1), (0, 0)))

    E = _build_gather_e()(entity_emb, e)
    x2e, rel_scores = _mlp(E, H, W1, b1.reshape(1, AD), W2, b2.reshape(1, AD),
                           relP)
    scores = _build_scores_sc()(e_space, r_space, action_mask, x2e,
                                rel_scores, entity_emb)
    dist, ent = _softmax(scores)
    return dist, ent.reshape(B)
